# Initial kernel scaffold; baseline (speedup 1.0000x reference)
import jax
import jax.numpy as jnp
from jax.experimental import pallas as pl


def kernel(inputs, embedding_weight):
    raise NotImplementedError("write your pallas kernel here")


# speedup vs baseline: 1.1033x; 1.1033x over previous
import functools

import jax
import jax.numpy as jnp
from jax import lax
from jax.experimental import pallas as pl
from jax.experimental.pallas import tpu as pltpu
from jax.experimental.pallas import tpu_sc as plsc

NC = 2
NS = 16
NW = NC * NS
D = 32
CH = 1024


@functools.lru_cache(maxsize=None)
def _gather_kernel(N):
    n_per_w = N // NW
    n_ch = n_per_w // CH
    mesh = plsc.VectorSubcoreMesh(
        core_axis_name="c", subcore_axis_name="s",
        num_cores=NC, num_subcores=NS)

    @functools.partial(
        pl.kernel,
        out_type=jax.ShapeDtypeStruct((N, D), jnp.float32),
        mesh=mesh,
        scratch_types=[
            pltpu.VMEM((n_per_w,), jnp.int32),
            pltpu.VMEM((CH, D), jnp.float32),
            pltpu.SemaphoreType.DMA,
        ],
        compiler_params=pltpu.CompilerParams(use_tc_tiling_on_sc=False),
    )
    def k(idx_hbm, table_hbm, out_hbm, idx_v, rows_v, sem):
        wid = lax.axis_index("s") * NC + lax.axis_index("c")
        base = wid * n_per_w
        pltpu.sync_copy(idx_hbm.at[pl.ds(base, n_per_w)], idx_v)

        def body(c, carry):
            off = c * CH
            pltpu.async_copy(
                table_hbm.at[idx_v.at[pl.ds(off, CH)]], rows_v, sem).wait()
            pltpu.sync_copy(rows_v, out_hbm.at[pl.ds(base + off, CH)])
            return carry

        lax.fori_loop(0, n_ch, body, 0)

    return k


def kernel(inputs, embedding_weight):
    B, H = inputs.shape
    N = B * H
    idx = inputs.reshape(N).astype(jnp.int32)
    out = _gather_kernel(N)(idx, embedding_weight)
    return out.reshape(B, H, D)


# speedup vs baseline: 1.1113x; 1.0072x over previous
import functools

import jax
import jax.numpy as jnp
from jax import lax
from jax.experimental import pallas as pl
from jax.experimental.pallas import tpu as pltpu
from jax.experimental.pallas import tpu_sc as plsc

NC = 2
NS = 16
NW = NC * NS
D = 32
CH = 640
NBUF = 4
SUPER = CH * NBUF


@functools.lru_cache(maxsize=None)
def _gather_kernel(N):
    n_per_w = N // NW
    n_super = n_per_w // SUPER
    mesh = plsc.VectorSubcoreMesh(
        core_axis_name="c", subcore_axis_name="s",
        num_cores=NC, num_subcores=NS)

    @functools.partial(
        pl.kernel,
        out_type=jax.ShapeDtypeStruct((N, D), jnp.float32),
        mesh=mesh,
        scratch_types=[
            pltpu.VMEM((n_per_w,), jnp.int32),
            pltpu.VMEM((SUPER, D), jnp.float32),
        ] + [pltpu.SemaphoreType.DMA] * NBUF,
        compiler_params=pltpu.CompilerParams(use_tc_tiling_on_sc=False),
    )
    def k(idx_hbm, table_hbm, out_hbm, idx_v, rows_v, *sems):
        wid = lax.axis_index("s") * NC + lax.axis_index("c")
        base = wid * n_per_w
        pltpu.sync_copy(idx_hbm.at[pl.ds(base, n_per_w)], idx_v)

        def body(g, carry):
            off = g * SUPER
            copies = []
            for b in range(NBUF):
                copies.append(pltpu.async_copy(
                    table_hbm.at[idx_v.at[pl.ds(off + b * CH, CH)]],
                    rows_v.at[pl.ds(b * CH, CH)],
                    sems[b]))
            for c in copies:
                c.wait()
            pltpu.sync_copy(rows_v, out_hbm.at[pl.ds(base + off, SUPER)])
            return carry

        lax.fori_loop(0, n_super, body, 0)

    return k


def kernel(inputs, embedding_weight):
    B, H = inputs.shape
    N = B * H
    idx = inputs.reshape(N).astype(jnp.int32)
    out = _gather_kernel(N)(idx, embedding_weight)
    return out.reshape(B, H, D)


# speedup vs baseline: 1.7995x; 1.6192x over previous
import functools

import jax
import jax.numpy as jnp
from jax import lax
from jax.experimental import pallas as pl
from jax.experimental.pallas import tpu as pltpu
from jax.experimental.pallas import tpu_sc as plsc

NC = 2
NS = 16
NW = NC * NS
D = 32
CH = 400
NBUF = 4
SUPER = CH * NBUF


@functools.lru_cache(maxsize=None)
def _gather_kernel(N):
    n_per_w = N // NW
    n_super = n_per_w // SUPER
    mesh = plsc.VectorSubcoreMesh(
        core_axis_name="c", subcore_axis_name="s",
        num_cores=NC, num_subcores=NS)

    H = 50
    cb = SUPER // H

    @functools.partial(
        pl.kernel,
        out_type=jax.ShapeDtypeStruct((N // H, H, D), jnp.float32),
        mesh=mesh,
        scratch_types=[
            pltpu.VMEM((n_per_w,), jnp.int32),
            pltpu.VMEM((SUPER, D), jnp.float32),
        ] + [pltpu.SemaphoreType.DMA] * (NBUF + 1),
        compiler_params=pltpu.CompilerParams(use_tc_tiling_on_sc=False),
    )
    def k(idx_hbm, table_hbm, out_hbm, idx_v, rows_v, *sems):
        wsem = sems[NBUF]
        wid = lax.axis_index("s") * NC + lax.axis_index("c")
        base = wid * n_per_w
        b_base = wid * (n_per_w // H)
        pltpu.sync_copy(idx_hbm.at[pl.ds(base, n_per_w)], idx_v)

        def body(g, carry):
            off = g * SUPER
            copies = []
            for b in range(NBUF):
                copies.append(pltpu.async_copy(
                    table_hbm.at[idx_v.at[pl.ds(off + b * CH, CH)]],
                    rows_v.at[pl.ds(b * CH, CH)],
                    sems[b]))
            for c in copies:
                c.wait()
            wb = []
            for i in range(cb):
                wb.append(pltpu.async_copy(
                    rows_v.at[pl.ds(i * H, H)],
                    out_hbm.at[b_base + g * cb + i], wsem))
            for c in wb:
                c.wait()
            return carry

        lax.fori_loop(0, n_super, body, 0)

    return k


def kernel(inputs, embedding_weight):
    B, H = inputs.shape
    N = B * H
    idx = inputs.reshape(N).astype(jnp.int32)
    return _gather_kernel(N)(idx, embedding_weight)
